# baseline (speedup 1.0000x reference)
import functools

import jax
import jax.numpy as jnp
from jax import lax
from jax.experimental import pallas as pl
from jax.experimental.pallas import tpu as pltpu
from jax.experimental.pallas import tpu_sc as plsc

NUM_CLASSES = 22
SEQ_LEN = 8192

_info = plsc.get_sparse_core_info()
_NC, _NS, _L = _info.num_cores, _info.num_subcores, _info.num_lanes
_NW = _NC * _NS
_TOK_PER_W = SEQ_LEN // _NW
_VECS = _TOK_PER_W // _L


@functools.partial(
    pl.kernel,
    mesh=plsc.VectorSubcoreMesh(core_axis_name="c", subcore_axis_name="s"),
    out_type=jax.ShapeDtypeStruct((NUM_CLASSES, SEQ_LEN), jnp.float32),
    scratch_types=[
        pltpu.VMEM((_TOK_PER_W,), jnp.int32),
        pltpu.VMEM((NUM_CLASSES, _TOK_PER_W), jnp.float32),
    ],
)
def _onehot_sc(x_hbm, out_hbm, x_v, blk_v):
    wid = lax.axis_index("s") * _NC + lax.axis_index("c")
    base = wid * _TOK_PER_W
    pltpu.sync_copy(x_hbm.at[pl.ds(base, _TOK_PER_W)], x_v)
    one = jnp.full((_L,), 1.0, dtype=jnp.float32)
    zero = jnp.zeros((_L,), dtype=jnp.float32)

    @plsc.parallel_loop(0, _TOK_PER_W, step=_L, unroll=4)
    def _(off):
        xv = x_v[pl.ds(off, _L)]
        for c in range(NUM_CLASSES):
            blk_v[c, pl.ds(off, _L)] = jnp.where(xv == c, one, zero)

    pltpu.sync_copy(blk_v, out_hbm.at[:, pl.ds(base, _TOK_PER_W)])


def kernel(x):
    return _onehot_sc(x.astype(jnp.int32)).reshape(1, NUM_CLASSES, SEQ_LEN)


# speedup vs baseline: 1.0012x; 1.0012x over previous
import functools

import jax
import jax.numpy as jnp
from jax import lax
from jax.experimental import pallas as pl
from jax.experimental.pallas import tpu as pltpu
from jax.experimental.pallas import tpu_sc as plsc

NUM_CLASSES = 22
SEQ_LEN = 8192

_info = plsc.get_sparse_core_info()
_NC, _NS, _L = _info.num_cores, _info.num_subcores, _info.num_lanes
_NW = _NC * _NS
_TOK_PER_W = SEQ_LEN // _NW
_VECS = _TOK_PER_W // _L


@functools.partial(
    pl.kernel,
    mesh=plsc.VectorSubcoreMesh(core_axis_name="c", subcore_axis_name="s"),
    out_type=jax.ShapeDtypeStruct((NUM_CLASSES, SEQ_LEN), jnp.float32),
    scratch_types=[
        pltpu.VMEM((_TOK_PER_W,), jnp.int32),
        pltpu.VMEM((NUM_CLASSES, _TOK_PER_W), jnp.float32),
    ],
)
def _onehot_sc(x_hbm, out_hbm, x_v, blk_v):
    wid = lax.axis_index("s") * _NC + lax.axis_index("c")
    base = wid * _TOK_PER_W
    pltpu.sync_copy(x_hbm.at[pl.ds(base, _TOK_PER_W)], x_v)
    one = jnp.full((_L,), 1.0, dtype=jnp.float32)
    zero = jnp.zeros((_L,), dtype=jnp.float32)

    @plsc.parallel_loop(0, _TOK_PER_W, step=_L, unroll=16)
    def _(off):
        xv = x_v[pl.ds(off, _L)]
        for c in range(NUM_CLASSES):
            blk_v[c, pl.ds(off, _L)] = jnp.where(xv == c, one, zero)

    pltpu.sync_copy(blk_v, out_hbm.at[:, pl.ds(base, _TOK_PER_W)])


def kernel(x):
    return _onehot_sc(x.astype(jnp.int32)).reshape(1, NUM_CLASSES, SEQ_LEN)


# speedup vs baseline: 1.0021x; 1.0009x over previous
import functools

import jax
import jax.numpy as jnp
from jax import lax
from jax.experimental import pallas as pl
from jax.experimental.pallas import tpu as pltpu
from jax.experimental.pallas import tpu_sc as plsc

NUM_CLASSES = 22
SEQ_LEN = 8192

_info = plsc.get_sparse_core_info()
_NC, _NS, _L = _info.num_cores, _info.num_subcores, _info.num_lanes
_NW = _NC * _NS
_TOK_PER_W = SEQ_LEN // _NW
_VECS = _TOK_PER_W // _L


@functools.partial(
    pl.kernel,
    mesh=plsc.VectorSubcoreMesh(core_axis_name="c", subcore_axis_name="s"),
    out_type=jax.ShapeDtypeStruct((1, NUM_CLASSES, SEQ_LEN), jnp.float32),
    scratch_types=[
        pltpu.VMEM((_TOK_PER_W,), jnp.int32),
        pltpu.VMEM((NUM_CLASSES, _TOK_PER_W), jnp.float32),
    ],
)
def _onehot_sc(x_hbm, out_hbm, x_v, blk_v):
    wid = lax.axis_index("s") * _NC + lax.axis_index("c")
    base = wid * _TOK_PER_W
    pltpu.sync_copy(x_hbm.at[pl.ds(base, _TOK_PER_W)], x_v)
    one = jnp.full((_L,), 1.0, dtype=jnp.float32)
    zero = jnp.zeros((_L,), dtype=jnp.float32)

    @plsc.parallel_loop(0, _TOK_PER_W, step=_L, unroll=16)
    def _(off):
        xv = x_v[pl.ds(off, _L)]
        for c in range(NUM_CLASSES):
            blk_v[c, pl.ds(off, _L)] = jnp.where(xv == c, one, zero)

    pltpu.sync_copy(blk_v, out_hbm.at[0, :, pl.ds(base, _TOK_PER_W)])


def kernel(x):
    return _onehot_sc(x.astype(jnp.int32))


# speedup vs baseline: 1.0048x; 1.0026x over previous
import functools

import jax
import jax.numpy as jnp
from jax import lax
from jax.experimental import pallas as pl
from jax.experimental.pallas import tpu as pltpu
from jax.experimental.pallas import tpu_sc as plsc

NUM_CLASSES = 22
SEQ_LEN = 8192

_info = plsc.get_sparse_core_info()
_NC, _NS, _L = _info.num_cores, _info.num_subcores, _info.num_lanes
_NW = _NC * _NS
_NQ = 4
_GRP = _NW // _NQ
_CHUNK = SEQ_LEN // _NQ


@functools.partial(
    pl.kernel,
    mesh=plsc.VectorSubcoreMesh(core_axis_name="c", subcore_axis_name="s"),
    out_type=jax.ShapeDtypeStruct((1, NUM_CLASSES, SEQ_LEN), jnp.float32),
    scratch_types=[
        pltpu.VMEM((_CHUNK,), jnp.int32),
        pltpu.VMEM((_CHUNK,), jnp.float32),
    ],
)
def _onehot_sc(x_hbm, out_hbm, x_v, row_v):
    wid = lax.axis_index("s") * _NC + lax.axis_index("c")
    q = wid // _GRP
    k = wid % _GRP
    base = q * _CHUNK
    pltpu.sync_copy(x_hbm.at[pl.ds(base, _CHUNK)], x_v)
    one = jnp.full((_L,), 1.0, dtype=jnp.float32)
    zero = jnp.zeros((_L,), dtype=jnp.float32)

    def do_row(row):
        @plsc.parallel_loop(0, _CHUNK, step=_L, unroll=8)
        def _(off):
            xv = x_v[pl.ds(off, _L)]
            row_v[pl.ds(off, _L)] = jnp.where(xv == row, one, zero)

        pltpu.sync_copy(row_v, out_hbm.at[0, row, pl.ds(base, _CHUNK)])

    do_row(k)
    do_row(k + _GRP)

    @pl.when(k + 2 * _GRP < NUM_CLASSES)
    def _():
        do_row(k + 2 * _GRP)


def kernel(x):
    return _onehot_sc(x.astype(jnp.int32))
